# initial kernel scaffold (unmeasured)
import jax
import jax.numpy as jnp
from jax import lax
from jax.experimental import pallas as pl
from jax.experimental.pallas import tpu as pltpu


def kernel(
    x,
):
    def body(*refs):
        pass

    out_shape = jax.ShapeDtypeStruct(..., jnp.float32)
    return pl.pallas_call(body, out_shape=out_shape)(...)


# baseline (device time: 32145 ns/iter reference)
import jax
import jax.numpy as jnp
from jax import lax
from jax.experimental import pallas as pl
from jax.experimental.pallas import tpu as pltpu


def kernel(x):
    m, n = x.shape

    def body(
        x_ref, out_ref, row_buf, col_buf, row_send, col_send, send_sems, recv_sems
    ):
        my_x = lax.axis_index("x")
        my_y = lax.axis_index("y")
        x_nbr = (1 - my_x, my_y)
        y_nbr = (my_x, 1 - my_y)

        barrier_sem = pltpu.get_barrier_semaphore()
        pl.semaphore_signal(
            barrier_sem, inc=1, device_id=x_nbr,
            device_id_type=pl.DeviceIdType.MESH,
        )
        pl.semaphore_signal(
            barrier_sem, inc=1, device_id=y_nbr,
            device_id_type=pl.DeviceIdType.MESH,
        )
        pl.semaphore_wait(barrier_sem, 2)

        @pl.when(my_x == 0)
        def _():
            row_send[:, :] = x_ref[m - 1 : m, :]

        @pl.when(my_x == 1)
        def _():
            row_send[:, :] = x_ref[0:1, :]

        @pl.when(my_y == 0)
        def _():
            col_send[:, :] = x_ref[:, n - 1 : n]

        @pl.when(my_y == 1)
        def _():
            col_send[:, :] = x_ref[:, 0:1]

        rdma_row = pltpu.make_async_remote_copy(
            src_ref=row_send,
            dst_ref=row_buf,
            send_sem=send_sems.at[0],
            recv_sem=recv_sems.at[0],
            device_id=x_nbr,
            device_id_type=pl.DeviceIdType.MESH,
        )
        rdma_row.start()
        rdma_col = pltpu.make_async_remote_copy(
            src_ref=col_send,
            dst_ref=col_buf,
            send_sem=send_sems.at[1],
            recv_sem=recv_sems.at[1],
            device_id=y_nbr,
            device_id_type=pl.DeviceIdType.MESH,
        )
        rdma_col.start()

        xv = x_ref[:, :].astype(jnp.bfloat16)

        rdma_row.wait()
        rdma_col.wait()

        halo_row = row_buf[:, :].astype(jnp.bfloat16)
        halo_col = col_buf[:, :].astype(jnp.bfloat16)

        zero_row = jnp.zeros((1, n), jnp.bfloat16)
        zero_col = jnp.zeros((m, 1), jnp.bfloat16)
        north = jnp.where(my_x == 1, halo_row, zero_row)
        south = jnp.where(my_x == 0, halo_row, zero_row)
        west = jnp.where(my_y == 1, halo_col, zero_col)
        east = jnp.where(my_y == 0, halo_col, zero_col)

        c_half = jnp.bfloat16(0.5)
        c_eighth = jnp.bfloat16(0.125)
        res = xv * c_half
        res = res + c_eighth * jnp.concatenate([north, xv[:-1, :]], axis=0)
        res = res + c_eighth * jnp.concatenate([xv[1:, :], south], axis=0)
        res = res + c_eighth * jnp.concatenate([west, xv[:, :-1]], axis=1)
        res = res + c_eighth * jnp.concatenate([xv[:, 1:], east], axis=1)
        out_ref[:, :] = res

        @pl.when(my_x == 0)
        def _():
            out_ref[0:1, :] = x_ref[0:1, :].astype(jnp.bfloat16)

        @pl.when(my_x == 1)
        def _():
            out_ref[m - 1 : m, :] = x_ref[m - 1 : m, :].astype(jnp.bfloat16)

        @pl.when(my_y == 0)
        def _():
            out_ref[:, 0:1] = x_ref[:, 0:1].astype(jnp.bfloat16)

        @pl.when(my_y == 1)
        def _():
            out_ref[:, n - 1 : n] = x_ref[:, n - 1 : n].astype(jnp.bfloat16)

    return pl.pallas_call(
        body,
        out_shape=jax.ShapeDtypeStruct((m, n), jnp.bfloat16),
        in_specs=[pl.BlockSpec(memory_space=pltpu.VMEM)],
        out_specs=pl.BlockSpec(memory_space=pltpu.VMEM),
        scratch_shapes=[
            pltpu.VMEM((1, n), x.dtype),
            pltpu.VMEM((m, 1), x.dtype),
            pltpu.VMEM((1, n), x.dtype),
            pltpu.VMEM((m, 1), x.dtype),
            pltpu.SemaphoreType.DMA((2,)),
            pltpu.SemaphoreType.DMA((2,)),
        ],
        compiler_params=pltpu.CompilerParams(
            collective_id=0, vmem_limit_bytes=40 * 1024 * 1024
        ),
    )(x)


# device time: 25788 ns/iter; 1.2465x vs baseline; 1.2465x over previous
import jax
import jax.numpy as jnp
from jax import lax
from jax.experimental import pallas as pl
from jax.experimental.pallas import tpu as pltpu


def kernel(x):
    m, n = x.shape

    def body(
        x_ref, out_ref, row_buf, col_buf, row_send, col_send, send_sems, recv_sems
    ):
        my_x = lax.axis_index("x")
        my_y = lax.axis_index("y")
        x_nbr = (1 - my_x, my_y)
        y_nbr = (my_x, 1 - my_y)

        barrier_sem = pltpu.get_barrier_semaphore()
        pl.semaphore_signal(
            barrier_sem, inc=1, device_id=x_nbr,
            device_id_type=pl.DeviceIdType.MESH,
        )
        pl.semaphore_signal(
            barrier_sem, inc=1, device_id=y_nbr,
            device_id_type=pl.DeviceIdType.MESH,
        )
        pl.semaphore_wait(barrier_sem, 2)

        @pl.when(my_x == 0)
        def _():
            row_send[:, :] = x_ref[m - 1 : m, :]

        @pl.when(my_x == 1)
        def _():
            row_send[:, :] = x_ref[0:1, :]

        @pl.when(my_y == 0)
        def _():
            col_send[:, :] = x_ref[:, n - 1 : n]

        @pl.when(my_y == 1)
        def _():
            col_send[:, :] = x_ref[:, 0:1]

        rdma_row = pltpu.make_async_remote_copy(
            src_ref=row_send,
            dst_ref=row_buf,
            send_sem=send_sems.at[0],
            recv_sem=recv_sems.at[0],
            device_id=x_nbr,
            device_id_type=pl.DeviceIdType.MESH,
        )
        rdma_row.start()
        rdma_col = pltpu.make_async_remote_copy(
            src_ref=col_send,
            dst_ref=col_buf,
            send_sem=send_sems.at[1],
            recv_sem=recv_sems.at[1],
            device_id=y_nbr,
            device_id_type=pl.DeviceIdType.MESH,
        )
        rdma_col.start()

        c_half = jnp.bfloat16(0.5)
        c_eighth = jnp.bfloat16(0.125)
        xv = x_ref[:, :].astype(jnp.bfloat16)
        s = pltpu.roll(xv, 1, 0) + pltpu.roll(xv, m - 1, 0)
        s = s + pltpu.roll(xv, 1, 1)
        s = s + pltpu.roll(xv, n - 1, 1)
        out_ref[:, :] = xv * c_half + s * c_eighth

        rdma_row.wait()
        rdma_col.wait()

        halo_row = row_buf[:, :].astype(jnp.bfloat16)
        halo_col = col_buf[:, :].astype(jnp.bfloat16)

        @pl.when(my_x == 1)
        def _():
            wrong = x_ref[m - 1 : m, :].astype(jnp.bfloat16)
            out_ref[0:1, :] = out_ref[0:1, :] + c_eighth * (halo_row - wrong)

        @pl.when(my_x == 0)
        def _():
            wrong = x_ref[0:1, :].astype(jnp.bfloat16)
            out_ref[m - 1 : m, :] = (
                out_ref[m - 1 : m, :] + c_eighth * (halo_row - wrong)
            )

        @pl.when(my_y == 1)
        def _():
            wrong = x_ref[:, n - 1 : n].astype(jnp.bfloat16)
            out_ref[:, 0:1] = out_ref[:, 0:1] + c_eighth * (halo_col - wrong)

        @pl.when(my_y == 0)
        def _():
            wrong = x_ref[:, 0:1].astype(jnp.bfloat16)
            out_ref[:, n - 1 : n] = (
                out_ref[:, n - 1 : n] + c_eighth * (halo_col - wrong)
            )

        @pl.when(my_x == 0)
        def _():
            out_ref[0:1, :] = x_ref[0:1, :].astype(jnp.bfloat16)

        @pl.when(my_x == 1)
        def _():
            out_ref[m - 1 : m, :] = x_ref[m - 1 : m, :].astype(jnp.bfloat16)

        @pl.when(my_y == 0)
        def _():
            out_ref[:, 0:1] = x_ref[:, 0:1].astype(jnp.bfloat16)

        @pl.when(my_y == 1)
        def _():
            out_ref[:, n - 1 : n] = x_ref[:, n - 1 : n].astype(jnp.bfloat16)

    return pl.pallas_call(
        body,
        out_shape=jax.ShapeDtypeStruct((m, n), jnp.bfloat16),
        in_specs=[pl.BlockSpec(memory_space=pltpu.VMEM)],
        out_specs=pl.BlockSpec(memory_space=pltpu.VMEM),
        scratch_shapes=[
            pltpu.VMEM((1, n), x.dtype),
            pltpu.VMEM((m, 1), x.dtype),
            pltpu.VMEM((1, n), x.dtype),
            pltpu.VMEM((m, 1), x.dtype),
            pltpu.SemaphoreType.DMA((2,)),
            pltpu.SemaphoreType.DMA((2,)),
        ],
        compiler_params=pltpu.CompilerParams(
            collective_id=0, vmem_limit_bytes=40 * 1024 * 1024
        ),
    )(x)


# device time: 20226 ns/iter; 1.5893x vs baseline; 1.2750x over previous
import jax
import jax.numpy as jnp
from jax import lax
from jax.experimental import pallas as pl
from jax.experimental.pallas import tpu as pltpu

T = 128


def kernel(x):
    m, n = x.shape

    def body(x_ref, out_ref, row_send, strip_send, row_buf, strip_buf,
             send_sems, recv_sems):
        my_x = lax.axis_index("x")
        my_y = lax.axis_index("y")
        x_nbr = (1 - my_x, my_y)
        y_nbr = (my_x, 1 - my_y)

        c_half = jnp.bfloat16(0.5)
        c_eighth = jnp.bfloat16(0.125)

        barrier_sem = pltpu.get_barrier_semaphore()
        pl.semaphore_signal(
            barrier_sem, inc=1, device_id=x_nbr,
            device_id_type=pl.DeviceIdType.MESH,
        )
        pl.semaphore_signal(
            barrier_sem, inc=1, device_id=y_nbr,
            device_id_type=pl.DeviceIdType.MESH,
        )
        pl.semaphore_wait(barrier_sem, 2)

        @pl.when(my_x == 0)
        def _():
            row_send[:, :] = x_ref[m - 1 : m, :]

        @pl.when(my_x == 1)
        def _():
            row_send[:, :] = x_ref[0:1, :]

        @pl.when(my_y == 0)
        def _():
            strip_send[:, :] = x_ref[:, n - T : n].astype(jnp.bfloat16)

        @pl.when(my_y == 1)
        def _():
            strip_send[:, :] = x_ref[:, 0:T].astype(jnp.bfloat16)

        rdma_row = pltpu.make_async_remote_copy(
            src_ref=row_send,
            dst_ref=row_buf,
            send_sem=send_sems.at[0],
            recv_sem=recv_sems.at[0],
            device_id=x_nbr,
            device_id_type=pl.DeviceIdType.MESH,
        )
        rdma_row.start()
        rdma_col = pltpu.make_async_remote_copy(
            src_ref=strip_send,
            dst_ref=strip_buf,
            send_sem=send_sems.at[1],
            recv_sem=recv_sems.at[1],
            device_id=y_nbr,
            device_id_type=pl.DeviceIdType.MESH,
        )
        rdma_col.start()

        xv = x_ref[:, :].astype(jnp.bfloat16)
        s = pltpu.roll(xv, 1, 0) + pltpu.roll(xv, m - 1, 0)
        s = s + pltpu.roll(xv, 1, 1)
        s = s + pltpu.roll(xv, n - 1, 1)
        out_ref[:, :] = xv * c_half + s * c_eighth

        rdma_row.wait()
        rdma_col.wait()
        halo_row = row_buf[:, :].astype(jnp.bfloat16)
        sb = strip_buf[:, :]

        @pl.when(my_y == 1)
        def _():
            strip_buf[:, :] = pltpu.roll(sb, 1, 1)

        halo_col = strip_buf[:, 0:1]

        @pl.when(my_x == 1)
        def _():
            wrong = x_ref[m - 1 : m, :].astype(jnp.bfloat16)
            out_ref[0:1, :] = out_ref[0:1, :] + c_eighth * (halo_row - wrong)

        @pl.when(my_x == 0)
        def _():
            wrong = x_ref[0:1, :].astype(jnp.bfloat16)
            out_ref[m - 1 : m, :] = (
                out_ref[m - 1 : m, :] + c_eighth * (halo_row - wrong)
            )

        @pl.when(my_y == 1)
        def _():
            wrong = x_ref[:, n - 1 : n].astype(jnp.bfloat16)
            out_ref[:, 0:1] = out_ref[:, 0:1] + c_eighth * (halo_col - wrong)

        @pl.when(my_y == 0)
        def _():
            wrong = x_ref[:, 0:1].astype(jnp.bfloat16)
            out_ref[:, n - 1 : n] = (
                out_ref[:, n - 1 : n] + c_eighth * (halo_col - wrong)
            )

        @pl.when(my_x == 0)
        def _():
            out_ref[0:1, :] = x_ref[0:1, :].astype(jnp.bfloat16)

        @pl.when(my_x == 1)
        def _():
            out_ref[m - 1 : m, :] = x_ref[m - 1 : m, :].astype(jnp.bfloat16)

        @pl.when(my_y == 0)
        def _():
            out_ref[:, 0:1] = x_ref[:, 0:1].astype(jnp.bfloat16)

        @pl.when(my_y == 1)
        def _():
            out_ref[:, n - 1 : n] = x_ref[:, n - 1 : n].astype(jnp.bfloat16)

    return pl.pallas_call(
        body,
        out_shape=jax.ShapeDtypeStruct((m, n), jnp.bfloat16),
        in_specs=[pl.BlockSpec(memory_space=pltpu.VMEM)],
        out_specs=pl.BlockSpec(memory_space=pltpu.VMEM),
        scratch_shapes=[
            pltpu.VMEM((1, n), x.dtype),
            pltpu.VMEM((m, T), jnp.bfloat16),
            pltpu.VMEM((1, n), x.dtype),
            pltpu.VMEM((m, T), jnp.bfloat16),
            pltpu.SemaphoreType.DMA((2,)),
            pltpu.SemaphoreType.DMA((2,)),
        ],
        compiler_params=pltpu.CompilerParams(
            collective_id=0, vmem_limit_bytes=40 * 1024 * 1024
        ),
    )(x)
